# baseline (speedup 1.0000x reference)
import functools
import math

import jax
import jax.numpy as jnp
from jax import lax
from jax.experimental import pallas as pl
from jax.experimental.pallas import tpu as pltpu
from jax.experimental.pallas import tpu_sc as plsc

D_MODEL = 64
VOCAB = 1000000
B, L = 16384, 50
SCALE = math.sqrt(D_MODEL)

NC, NS, LANES = 2, 16, 16
NW = NC * NS
BB = 128
N_BLOCKS = B // BB
BLK_PER_W = N_BLOCKS // NW
NGRP = BB // LANES


def _emb_body(xt_hbm, lut_hbm, out_hbm, i50_v, g0_v, g1_v, t0_v, t1_v,
              sg0, sg1, so0, so1):
    wid = lax.axis_index("s") * NC + lax.axis_index("c")
    g_bufs = (g0_v, g1_v)
    t_bufs = (t0_v, t1_v)
    g_sems = (sg0, sg1)
    o_sems = (so0, so1)

    def fire(l, buf):
        pltpu.async_copy(lut_hbm.at[i50_v.at[l]], g_bufs[buf], g_sems[buf])

    def wait_gather(buf):
        pltpu.make_async_copy(
            lut_hbm.at[i50_v.at[0]], g_bufs[buf], g_sems[buf]
        ).wait()

    def blk_body(bi, carry):
        bb0 = (wid * BLK_PER_W + bi) * BB
        pltpu.sync_copy(xt_hbm.at[:, pl.ds(bb0, BB)], i50_v)
        fire(0, 0)

        def wait_out(buf):
            pltpu.make_async_copy(
                t_bufs[buf], out_hbm.at[0, :, pl.ds(bb0, BB)], o_sems[buf]
            ).wait()

        def l_body(l2, c2):
            for p in range(2):
                l = l2 * 2 + p
                buf = p
                nxt = jnp.minimum(l + 1, L - 1)
                fire(nxt, 1 - buf)
                wait_gather(buf)
                pl.when(l2 > 0)(lambda b=buf: wait_out(b))
                gv = g_bufs[buf]
                tv = t_bufs[buf]
                for grp in range(NGRP):
                    ridx = jax.lax.iota(jnp.int32, LANES) + grp * LANES
                    for c in range(D_MODEL):
                        cidx = jnp.full((LANES,), c, jnp.int32)
                        tv[c, pl.ds(grp * LANES, LANES)] = (
                            plsc.load_gather(gv, [ridx, cidx]) * SCALE
                        )
                pltpu.async_copy(
                    tv, out_hbm.at[l, :, pl.ds(bb0, BB)], o_sems[buf]
                )
            return c2

        lax.fori_loop(0, L // 2, l_body, 0)
        for buf in range(2):
            wait_out(buf)
        wait_gather(0)
        return carry

    lax.fori_loop(0, BLK_PER_W, blk_body, 0)


_emb = functools.partial(
    pl.kernel,
    mesh=plsc.VectorSubcoreMesh(core_axis_name="c", subcore_axis_name="s"),
    out_type=jax.ShapeDtypeStruct((L, D_MODEL, B), jnp.float32),
    scratch_types=[
        pltpu.VMEM((L, BB), jnp.int32),
        pltpu.VMEM((BB, 2 * D_MODEL), jnp.float32),
        pltpu.VMEM((BB, 2 * D_MODEL), jnp.float32),
        pltpu.VMEM((D_MODEL, BB), jnp.float32),
        pltpu.VMEM((D_MODEL, BB), jnp.float32),
        pltpu.SemaphoreType.DMA,
        pltpu.SemaphoreType.DMA,
        pltpu.SemaphoreType.DMA,
        pltpu.SemaphoreType.DMA,
    ],
    compiler_params=pltpu.CompilerParams(
        use_tc_tiling_on_sc=True, needs_layout_passes=False
    ),
)(_emb_body)


def kernel(x, lut):
    lutp = jnp.pad(lut, ((0, 0), (0, D_MODEL)))
    res = _emb(x.T, lutp)
    return jnp.transpose(res, (2, 0, 1))


# speedup vs baseline: 1.7349x; 1.7349x over previous
import functools
import math

import jax
import jax.numpy as jnp
from jax import lax
from jax.experimental import pallas as pl
from jax.experimental.pallas import tpu as pltpu
from jax.experimental.pallas import tpu_sc as plsc

D_MODEL = 64
VOCAB = 1000000
B, L = 16384, 50
SCALE = math.sqrt(D_MODEL)

NC, NS, LANES = 2, 16, 16
NW = NC * NS
BB = 128
N_BLOCKS = B // BB
BLK_PER_W = N_BLOCKS // NW
NGRP = BB // LANES


def _emb_body(xt_hbm, lut_hbm, out_hbm, i50_v, g0_v, g1_v, t0_v, t1_v,
              sg0, sg1, so0, so1):
    wid = lax.axis_index("s") * NC + lax.axis_index("c")
    g_bufs = (g0_v, g1_v)
    t_bufs = (t0_v, t1_v)
    g_sems = (sg0, sg1)
    o_sems = (so0, so1)

    def fire(l, buf):
        pltpu.async_copy(lut_hbm.at[i50_v.at[l]], g_bufs[buf], g_sems[buf])

    def wait_gather(buf):
        pltpu.make_async_copy(
            lut_hbm.at[i50_v.at[0]], g_bufs[buf], g_sems[buf]
        ).wait()

    def blk_body(bi, carry):
        bb0 = (wid * BLK_PER_W + bi) * BB
        pltpu.sync_copy(xt_hbm.at[:, pl.ds(bb0, BB)], i50_v)
        fire(0, 0)

        def wait_out(buf):
            pltpu.make_async_copy(
                t_bufs[buf], out_hbm.at[0, :, pl.ds(bb0, BB)], o_sems[buf]
            ).wait()

        def l_body(l2, c2):
            for p in range(2):
                l = l2 * 2 + p
                buf = p
                nxt = jnp.minimum(l + 1, L - 1)
                fire(nxt, 1 - buf)
                wait_gather(buf)
                pl.when(l2 > 0)(lambda b=buf: wait_out(b))
                gv = g_bufs[buf]
                tv = t_bufs[buf]
                ridxs = [
                    jax.lax.iota(jnp.int32, LANES) + grp * LANES
                    for grp in range(NGRP)
                ]

                @plsc.parallel_loop(0, D_MODEL, unroll=4)
                def c_body(c):
                    cidx = jnp.full((LANES,), c, jnp.int32)
                    for grp in range(NGRP):
                        tv[c, pl.ds(grp * LANES, LANES)] = (
                            plsc.load_gather(gv, [ridxs[grp], cidx]) * SCALE
                        )
                pltpu.async_copy(
                    tv, out_hbm.at[l, :, pl.ds(bb0, BB)], o_sems[buf]
                )
            return c2

        lax.fori_loop(0, L // 2, l_body, 0)
        for buf in range(2):
            wait_out(buf)
        wait_gather(0)
        return carry

    lax.fori_loop(0, BLK_PER_W, blk_body, 0)


_emb = functools.partial(
    pl.kernel,
    mesh=plsc.VectorSubcoreMesh(core_axis_name="c", subcore_axis_name="s"),
    out_type=jax.ShapeDtypeStruct((L, D_MODEL, B), jnp.float32),
    scratch_types=[
        pltpu.VMEM((L, BB), jnp.int32),
        pltpu.VMEM((BB, 2 * D_MODEL), jnp.float32),
        pltpu.VMEM((BB, 2 * D_MODEL), jnp.float32),
        pltpu.VMEM((D_MODEL, BB), jnp.float32),
        pltpu.VMEM((D_MODEL, BB), jnp.float32),
        pltpu.SemaphoreType.DMA,
        pltpu.SemaphoreType.DMA,
        pltpu.SemaphoreType.DMA,
        pltpu.SemaphoreType.DMA,
    ],
    compiler_params=pltpu.CompilerParams(
        use_tc_tiling_on_sc=True, needs_layout_passes=False
    ),
)(_emb_body)


def kernel(x, lut):
    lutp = jnp.pad(lut, ((0, 0), (0, D_MODEL)))
    res = _emb(x.T, lutp)
    return jnp.transpose(res, (2, 0, 1))


# speedup vs baseline: 1.7401x; 1.0030x over previous
import functools
import math

import jax
import jax.numpy as jnp
from jax import lax
from jax.experimental import pallas as pl
from jax.experimental.pallas import tpu as pltpu
from jax.experimental.pallas import tpu_sc as plsc

D_MODEL = 64
VOCAB = 1000000
B, L = 16384, 50
SCALE = math.sqrt(D_MODEL)

NC, NS, LANES = 2, 16, 16
NW = NC * NS
BB = 128
N_BLOCKS = B // BB
BLK_PER_W = N_BLOCKS // NW
NGRP = BB // LANES


def _emb_body(xt_hbm, lut_hbm, out_hbm, i50_v, g0_v, g1_v, t0_v, t1_v,
              sg0, sg1, so0, so1):
    wid = lax.axis_index("s") * NC + lax.axis_index("c")
    g_bufs = (g0_v, g1_v)
    t_bufs = (t0_v, t1_v)
    g_sems = (sg0, sg1)
    o_sems = (so0, so1)

    def fire(l, buf):
        pltpu.async_copy(lut_hbm.at[i50_v.at[l]], g_bufs[buf], g_sems[buf])

    def wait_gather(buf):
        pltpu.make_async_copy(
            lut_hbm.at[i50_v.at[0]], g_bufs[buf], g_sems[buf]
        ).wait()

    def blk_body(bi, carry):
        bb0 = (wid * BLK_PER_W + bi) * BB
        pltpu.sync_copy(xt_hbm.at[:, pl.ds(bb0, BB)], i50_v)
        fire(0, 0)

        def wait_out(buf):
            pltpu.make_async_copy(
                t_bufs[buf], out_hbm.at[0, :, pl.ds(bb0, BB)], o_sems[buf]
            ).wait()

        def l_body(l2, c2):
            for p in range(2):
                l = l2 * 2 + p
                buf = p
                nxt = jnp.minimum(l + 1, L - 1)
                fire(nxt, 1 - buf)
                wait_gather(buf)
                pl.when(l2 > 0)(lambda b=buf: wait_out(b))
                gv = g_bufs[buf]
                tv = t_bufs[buf]
                ridxs = [
                    jax.lax.iota(jnp.int32, LANES) + grp * LANES
                    for grp in range(NGRP)
                ]

                @plsc.parallel_loop(0, D_MODEL, unroll=8)
                def c_body(c):
                    cidx = jnp.full((LANES,), c, jnp.int32)
                    for grp in range(NGRP):
                        tv[c, pl.ds(grp * LANES, LANES)] = (
                            plsc.load_gather(gv, [ridxs[grp], cidx]) * SCALE
                        )
                pltpu.async_copy(
                    tv, out_hbm.at[l, :, pl.ds(bb0, BB)], o_sems[buf]
                )
            return c2

        lax.fori_loop(0, L // 2, l_body, 0)
        for buf in range(2):
            wait_out(buf)
        wait_gather(0)
        return carry

    lax.fori_loop(0, BLK_PER_W, blk_body, 0)


_emb = functools.partial(
    pl.kernel,
    mesh=plsc.VectorSubcoreMesh(core_axis_name="c", subcore_axis_name="s"),
    out_type=jax.ShapeDtypeStruct((L, D_MODEL, B), jnp.float32),
    scratch_types=[
        pltpu.VMEM((L, BB), jnp.int32),
        pltpu.VMEM((BB, 2 * D_MODEL), jnp.float32),
        pltpu.VMEM((BB, 2 * D_MODEL), jnp.float32),
        pltpu.VMEM((D_MODEL, BB), jnp.float32),
        pltpu.VMEM((D_MODEL, BB), jnp.float32),
        pltpu.SemaphoreType.DMA,
        pltpu.SemaphoreType.DMA,
        pltpu.SemaphoreType.DMA,
        pltpu.SemaphoreType.DMA,
    ],
    compiler_params=pltpu.CompilerParams(
        use_tc_tiling_on_sc=True, needs_layout_passes=False
    ),
)(_emb_body)


def kernel(x, lut):
    lutp = jnp.pad(lut, ((0, 0), (0, D_MODEL)))
    res = _emb(x.T, lutp)
    return jnp.transpose(res, (2, 0, 1))


# speedup vs baseline: 1.7476x; 1.0043x over previous
import functools
import math

import jax
import jax.numpy as jnp
from jax import lax
from jax.experimental import pallas as pl
from jax.experimental.pallas import tpu as pltpu
from jax.experimental.pallas import tpu_sc as plsc

D_MODEL = 64
VOCAB = 1000000
B, L = 16384, 50
SCALE = math.sqrt(D_MODEL)

NC, NS, LANES = 2, 16, 16
NW = NC * NS
BB = 256
NSUB = BB // 128
N_BLOCKS = B // BB
BLK_PER_W = N_BLOCKS // NW
NGRP = BB // LANES


def _emb_body(xt_hbm, lut_hbm, out_hbm, i50_v, g0_v, g1_v, t0_v, t1_v,
              sg0, sg1, so0, so1):
    wid = lax.axis_index("s") * NC + lax.axis_index("c")
    g_bufs = (g0_v, g1_v)
    t_bufs = (t0_v, t1_v)
    g_sems = (sg0, sg1)
    o_sems = (so0, so1)

    def fire(l, buf):
        for j in range(NSUB):
            pltpu.async_copy(
                lut_hbm.at[i50_v.at[l, j]],
                g_bufs[buf].at[pl.ds(j * 128, 128), :],
                g_sems[buf],
            )

    def wait_gather(buf):
        pltpu.make_async_copy(
            lut_hbm.at[i50_v.at[0, 0]],
            g_bufs[buf].at[pl.ds(0, 128), :],
            g_sems[buf],
        ).wait()
        pltpu.make_async_copy(
            lut_hbm.at[i50_v.at[0, 0]],
            g_bufs[buf].at[pl.ds(128, 128), :],
            g_sems[buf],
        ).wait()

    def blk_body(bi, carry):
        bb0 = (wid * BLK_PER_W + bi) * BB
        for j in range(NSUB):
            pltpu.sync_copy(
                xt_hbm.at[:, pl.ds(bb0 + j * 128, 128)], i50_v.at[:, j]
            )
        fire(0, 0)

        def wait_out(buf):
            pltpu.make_async_copy(
                t_bufs[buf], out_hbm.at[0, :, pl.ds(bb0, BB)], o_sems[buf]
            ).wait()

        def l_body(l2, c2):
            for p in range(2):
                l = l2 * 2 + p
                buf = p
                nxt = jnp.minimum(l + 1, L - 1)
                fire(nxt, 1 - buf)
                wait_gather(buf)
                pl.when(l2 > 0)(lambda b=buf: wait_out(b))
                gv = g_bufs[buf]
                tv = t_bufs[buf]
                ridxs = [
                    jax.lax.iota(jnp.int32, LANES) + grp * LANES
                    for grp in range(NGRP)
                ]

                @plsc.parallel_loop(0, D_MODEL, unroll=8)
                def c_body(c):
                    cidx = jnp.full((LANES,), c, jnp.int32)
                    for grp in range(NGRP):
                        tv[c, pl.ds(grp * LANES, LANES)] = (
                            plsc.load_gather(gv, [ridxs[grp], cidx]) * SCALE
                        )
                pltpu.async_copy(
                    tv, out_hbm.at[l, :, pl.ds(bb0, BB)], o_sems[buf]
                )
            return c2

        lax.fori_loop(0, L // 2, l_body, 0)
        for buf in range(2):
            wait_out(buf)
        wait_gather(0)
        return carry

    lax.fori_loop(0, BLK_PER_W, blk_body, 0)


_emb = functools.partial(
    pl.kernel,
    mesh=plsc.VectorSubcoreMesh(core_axis_name="c", subcore_axis_name="s"),
    out_type=jax.ShapeDtypeStruct((L, D_MODEL, B), jnp.float32),
    scratch_types=[
        pltpu.VMEM((L, NSUB, 128), jnp.int32),
        pltpu.VMEM((BB, 2 * D_MODEL), jnp.float32),
        pltpu.VMEM((BB, 2 * D_MODEL), jnp.float32),
        pltpu.VMEM((D_MODEL, BB), jnp.float32),
        pltpu.VMEM((D_MODEL, BB), jnp.float32),
        pltpu.SemaphoreType.DMA,
        pltpu.SemaphoreType.DMA,
        pltpu.SemaphoreType.DMA,
        pltpu.SemaphoreType.DMA,
    ],
    compiler_params=pltpu.CompilerParams(
        use_tc_tiling_on_sc=True, needs_layout_passes=False
    ),
)(_emb_body)


def kernel(x, lut):
    lutp = jnp.pad(lut, ((0, 0), (0, D_MODEL)))
    res = _emb(x.T, lutp)
    return jnp.transpose(res, (2, 0, 1))
